# baseline (speedup 1.0000x reference)
import math

import jax
import jax.numpy as jnp
from jax.experimental import pallas as pl

_INV_SQRT_PI = 1.0 / math.sqrt(math.pi)
_CUTOFF = 3.0

_ROWS = 8192
_COLS = 4096
_BLOCK_ROWS = 256


def _body(bm_ref, c_ref, bv_ref, o_ref):
    c = c_ref[...]
    arg = (bm_ref[...] - bv_ref[...]) * c
    absorption = jnp.exp(-(arg * arg)) * (c * jnp.float32(_INV_SQRT_PI))
    o_ref[...] = jnp.where(jnp.abs(arg) <= jnp.float32(_CUTOFF), absorption,
                           jnp.float32(0.0))


@jax.jit
def kernel(B_mean, c_extended, B_val):
    grid = (_ROWS // _BLOCK_ROWS,)
    return pl.pallas_call(
        _body,
        grid=grid,
        in_specs=[
            pl.BlockSpec((_BLOCK_ROWS, _COLS), lambda i: (i, 0)),
            pl.BlockSpec((_BLOCK_ROWS, 1), lambda i: (i, 0)),
            pl.BlockSpec((_BLOCK_ROWS, _COLS), lambda i: (i, 0)),
        ],
        out_specs=pl.BlockSpec((_BLOCK_ROWS, _COLS), lambda i: (i, 0)),
        out_shape=jax.ShapeDtypeStruct((_ROWS, _COLS), jnp.float32),
    )(B_mean, c_extended, B_val)
